# baseline (speedup 1.0000x reference)
import functools

import jax
import jax.numpy as jnp
from jax import lax
from jax.experimental import pallas as pl
from jax.experimental.pallas import tpu as pltpu
from jax.experimental.pallas import tpu_sc as plsc

N = 10000
E = 320000
D_IN = 128
HID = 16
NUM_CLASSES = 64

NC = 2
NS = 16
NW = NC * NS
EPW = E // NW
CHUNK = 125
NCHUNK = EPW // CHUNK
NBUF = 8
DEG_Q = 8

SLAB = 640
STRIDE = 624
NVB = SLAB // 16

_mesh = plsc.VectorSubcoreMesh(
    core_axis_name="c", subcore_axis_name="s", num_cores=NC, num_subcores=NS)

_sc_params = pltpu.CompilerParams(use_tc_tiling_on_sc=False)


def _rsqrt16(d):
    i = lax.bitcast_convert_type(d, jnp.int32)
    y = lax.bitcast_convert_type(jnp.int32(0x5F3759DF) - (i >> 1), jnp.float32)
    for _ in range(3):
        y = y * (1.5 - 0.5 * d * y * y)
    return y


@functools.partial(
    pl.kernel,
    out_type=jax.ShapeDtypeStruct((NC, N), jnp.float32),
    mesh=_mesh,
    scratch_types=[
        pltpu.VMEM((NCHUNK, CHUNK), jnp.int32),
        pltpu.VMEM((128,), jnp.float32),
        pltpu.VMEM_SHARED((N,), jnp.float32),
        pltpu.SemaphoreType.DMA,
    ],
)
def _deg_sc(dst_hbm, zeros1_hbm, out_hbm, dst_v, ones_v, deg_sh, sem):
    c = lax.axis_index("c")
    s = lax.axis_index("s")
    for i in range(128 // 16):
        ones_v[pl.ds(i * 16, 16)] = jnp.ones((16,), jnp.float32)

    @pl.when(s == 0)
    def _zero():
        pltpu.sync_copy(zeros1_hbm, deg_sh)

    plsc.subcore_barrier()
    base = (c * NS + s) * NCHUNK
    pltpu.sync_copy(dst_hbm.at[pl.ds(base, NCHUNK)], dst_v)

    def body(j, carry):
        pltpu.async_copy(ones_v.at[pl.ds(0, CHUNK)], deg_sh.at[dst_v.at[j]],
                         sem, add=True)

        @pl.when(j >= DEG_Q)
        def _pace():
            pltpu.make_async_copy(
                ones_v.at[pl.ds(0, CHUNK)], deg_sh.at[dst_v.at[j]], sem).wait()

        return carry

    lax.fori_loop(0, NCHUNK, body, 0)
    for _ in range(DEG_Q):
        pltpu.make_async_copy(
            ones_v.at[pl.ds(0, CHUNK)], deg_sh.at[dst_v.at[0]], sem).wait()
    plsc.subcore_barrier()

    @pl.when(s == 0)
    def _out():
        pltpu.sync_copy(deg_sh, out_hbm.at[c])


def _edge_phase(table, src_v, dst_v, rows_v, agg_sh, gsems, ssems):
    for b in range(NBUF):
        pltpu.async_copy(table.at[src_v.at[b]], rows_v.at[b], gsems[b])

    def body(o, carry):
        for b in range(NBUF):
            m = o * NBUF + b
            pltpu.make_async_copy(
                table.at[src_v.at[m]], rows_v.at[b], gsems[b]).wait()
            pltpu.async_copy(
                rows_v.at[b], agg_sh.at[dst_v.at[m]], ssems[b], add=True)

            @pl.when(o < NCHUNK // NBUF - 1)
            def _next():
                pltpu.make_async_copy(
                    rows_v.at[b], agg_sh.at[dst_v.at[m]], ssems[b]).wait()
                pltpu.async_copy(
                    table.at[src_v.at[m + NBUF]], rows_v.at[b], gsems[b])

        return carry

    lax.fori_loop(0, NCHUNK // NBUF, body, 0)
    for b in range(NBUF):
        m = NCHUNK - NBUF + b
        pltpu.make_async_copy(
            rows_v.at[b], agg_sh.at[dst_v.at[m]], ssems[b]).wait()


@functools.partial(
    pl.kernel,
    out_type=(
        jax.ShapeDtypeStruct((NC, N, HID), jnp.float32),
        jax.ShapeDtypeStruct((NC, N, HID), jnp.float32),
    ),
    mesh=_mesh,
    scratch_types=[
        pltpu.VMEM((SLAB, HID), jnp.float32),
        pltpu.VMEM((SLAB,), jnp.float32),
        pltpu.VMEM((SLAB,), jnp.float32),
        pltpu.VMEM((NCHUNK, CHUNK), jnp.int32),
        pltpu.VMEM((NCHUNK, CHUNK), jnp.int32),
        pltpu.VMEM((NBUF, CHUNK, HID), jnp.float32),
        pltpu.VMEM_SHARED((N, HID), jnp.float32),
        pltpu.VMEM_SHARED((N, HID), jnp.float32),
        [pltpu.SemaphoreType.DMA] * NBUF,
        [pltpu.SemaphoreType.DMA] * NBUF,
    ],
    compiler_params=_sc_params,
)
def _agg1_sc(hbar_hbm, degp_hbm, src_hbm, dst_hbm, zeros2_hbm,
             hs1d_hbm, out_hbm,
             hb_v, d0_v, d1_v, src_v, dst_v, rows_v, agg_sh, tb_sh,
             gsems, ssems):
    c = lax.axis_index("c")
    s = lax.axis_index("s")
    rbase = s * STRIDE
    base = (c * NS + s) * NCHUNK
    pltpu.sync_copy(src_hbm.at[pl.ds(base, NCHUNK)], src_v)
    pltpu.sync_copy(dst_hbm.at[pl.ds(base, NCHUNK)], dst_v)
    pltpu.sync_copy(zeros2_hbm.at[pl.ds(rbase, SLAB)],
                    agg_sh.at[pl.ds(rbase, SLAB)])
    pltpu.sync_copy(hbar_hbm.at[pl.ds(rbase, SLAB)], hb_v)
    pltpu.sync_copy(degp_hbm.at[0, pl.ds(rbase, SLAB)], d0_v)
    pltpu.sync_copy(degp_hbm.at[1, pl.ds(rbase, SLAB)], d1_v)

    def blk(b, carry):
        o = b * 16
        deg = d0_v[pl.ds(o, 16)] + d1_v[pl.ds(o, 16)] + 1.0
        dis = _rsqrt16(deg)
        for i in range(16):
            hb_v[o + i] = hb_v[o + i] * dis[i]
        return carry

    lax.fori_loop(0, NVB, blk, 0)
    pltpu.sync_copy(hb_v, hs1d_hbm.at[c, pl.ds(rbase, SLAB)])
    pltpu.sync_copy(hb_v, tb_sh.at[pl.ds(rbase, SLAB)])
    plsc.subcore_barrier()

    _edge_phase(tb_sh, src_v, dst_v, rows_v, agg_sh, gsems, ssems)
    plsc.subcore_barrier()
    pltpu.sync_copy(agg_sh.at[pl.ds(rbase, SLAB)],
                    out_hbm.at[c, pl.ds(rbase, SLAB)])


@functools.partial(
    pl.kernel,
    out_type=jax.ShapeDtypeStruct((NC, N, HID), jnp.float32),
    mesh=_mesh,
    scratch_types=[
        pltpu.VMEM((SLAB, HID), jnp.float32),
        pltpu.VMEM((SLAB, HID), jnp.float32),
        pltpu.VMEM((SLAB, HID), jnp.float32),
        pltpu.VMEM((SLAB,), jnp.float32),
        pltpu.VMEM((SLAB,), jnp.float32),
        pltpu.VMEM((HID,), jnp.float32),
        pltpu.VMEM((NCHUNK, CHUNK), jnp.int32),
        pltpu.VMEM((NCHUNK, CHUNK), jnp.int32),
        pltpu.VMEM((NBUF, CHUNK, HID), jnp.float32),
        pltpu.VMEM_SHARED((N, HID), jnp.float32),
        pltpu.VMEM_SHARED((N, HID), jnp.float32),
        [pltpu.SemaphoreType.DMA] * NBUF,
        [pltpu.SemaphoreType.DMA] * NBUF,
    ],
    compiler_params=_sc_params,
)
def _agg2_sc(agg1_hbm, hs1d_hbm, degp_hbm, b1_hbm, src_hbm, dst_hbm,
             zeros2_hbm, out_hbm,
             p0_v, p1_v, hs1_v, d0_v, d1_v, b1_v, src_v, dst_v, rows_v,
             agg_sh, tb_sh, gsems, ssems):
    c = lax.axis_index("c")
    s = lax.axis_index("s")
    rbase = s * STRIDE
    base = (c * NS + s) * NCHUNK
    pltpu.sync_copy(src_hbm.at[pl.ds(base, NCHUNK)], src_v)
    pltpu.sync_copy(dst_hbm.at[pl.ds(base, NCHUNK)], dst_v)
    pltpu.sync_copy(zeros2_hbm.at[pl.ds(rbase, SLAB)],
                    agg_sh.at[pl.ds(rbase, SLAB)])
    pltpu.sync_copy(agg1_hbm.at[0, pl.ds(rbase, SLAB)], p0_v)
    pltpu.sync_copy(agg1_hbm.at[1, pl.ds(rbase, SLAB)], p1_v)
    pltpu.sync_copy(hs1d_hbm.at[c, pl.ds(rbase, SLAB)], hs1_v)
    pltpu.sync_copy(degp_hbm.at[0, pl.ds(rbase, SLAB)], d0_v)
    pltpu.sync_copy(degp_hbm.at[1, pl.ds(rbase, SLAB)], d1_v)
    pltpu.sync_copy(b1_hbm, b1_v)
    b1r = b1_v[...]

    def blk(b, carry):
        o = b * 16
        deg = d0_v[pl.ds(o, 16)] + d1_v[pl.ds(o, 16)] + 1.0
        dis = _rsqrt16(deg)
        for i in range(16):
            n = o + i
            row = (p0_v[n] + p1_v[n] + hs1_v[n]) * dis[i] + b1r
            p0_v[n] = jnp.maximum(row, 0.0) * dis[i]
        return carry

    lax.fori_loop(0, NVB, blk, 0)
    pltpu.sync_copy(p0_v, tb_sh.at[pl.ds(rbase, SLAB)])
    plsc.subcore_barrier()

    _edge_phase(tb_sh, src_v, dst_v, rows_v, agg_sh, gsems, ssems)
    plsc.subcore_barrier()

    pltpu.sync_copy(agg_sh.at[pl.ds(rbase, SLAB)], hs1_v)

    def qblk(b, carry):
        o = b * 16
        deg = d0_v[pl.ds(o, 16)] + d1_v[pl.ds(o, 16)] + 1.0
        dis = _rsqrt16(deg)
        for i in range(16):
            n = o + i
            hs1_v[n] = (hs1_v[n] + 0.5 * p0_v[n]) * dis[i]
        return carry

    lax.fori_loop(0, NVB, qblk, 0)
    pltpu.sync_copy(hs1_v, out_hbm.at[c, pl.ds(rbase, SLAB)])


def _mm_body(x_ref, w1_ref, hb_ref):
    hb_ref[...] = jnp.dot(x_ref[...], w1_ref[...],
                          preferred_element_type=jnp.float32)


def _tc_mm(x, W1):
    return pl.pallas_call(
        _mm_body,
        out_shape=jax.ShapeDtypeStruct((N, HID), jnp.float32),
    )(x, W1)


def _out_body(q_ref, w2_ref, b2_ref, out_ref):
    a = q_ref[0] + q_ref[1]
    o = jnp.dot(a, w2_ref[...], preferred_element_type=jnp.float32)
    o = o + b2_ref[...]
    m = jnp.max(o, axis=1, keepdims=True)
    e = jnp.exp(o - m)
    lse = jnp.log(jnp.sum(e, axis=1, keepdims=True))
    out_ref[...] = (o - m) - lse


def _tc_out(q, W2, b2):
    return pl.pallas_call(
        _out_body,
        out_shape=jax.ShapeDtypeStruct((N, NUM_CLASSES), jnp.float32),
    )(q, W2, b2)


def kernel(x, edge_index, W1, b1, W2, b2):
    src2d = edge_index[0].reshape(E // CHUNK, CHUNK)
    dst2d = edge_index[1].reshape(E // CHUNK, CHUNK)
    zeros1 = jnp.zeros((N,), jnp.float32)
    zeros2 = jnp.zeros((N, HID), jnp.float32)

    hbar = _tc_mm(x, W1)
    degp = _deg_sc(dst2d, zeros1)
    hs1d, agg1 = _agg1_sc(hbar, degp, src2d, dst2d, zeros2)
    q = _agg2_sc(agg1, hs1d, degp, b1, src2d, dst2d, zeros2)
    return _tc_out(q, W2, b2.reshape(1, NUM_CLASSES))


# speedup vs baseline: 1.0463x; 1.0463x over previous
import functools

import jax
import jax.numpy as jnp
from jax import lax
from jax.experimental import pallas as pl
from jax.experimental.pallas import tpu as pltpu
from jax.experimental.pallas import tpu_sc as plsc

N = 10000
E = 320000
D_IN = 128
HID = 16
NUM_CLASSES = 64

NC = 2
NS = 16
NW = NC * NS
EPW = E // NW
CHUNK = 125
NCHUNK = EPW // CHUNK
NBUF = 8
DEG_Q = 8

SLAB = 640
STRIDE = 624
NVB = SLAB // 16

_mesh = plsc.VectorSubcoreMesh(
    core_axis_name="c", subcore_axis_name="s", num_cores=NC, num_subcores=NS)

_sc_params = pltpu.CompilerParams(use_tc_tiling_on_sc=False)


def _rsqrt16(d):
    i = lax.bitcast_convert_type(d, jnp.int32)
    y = lax.bitcast_convert_type(jnp.int32(0x5F3759DF) - (i >> 1), jnp.float32)
    for _ in range(3):
        y = y * (1.5 - 0.5 * d * y * y)
    return y


@functools.partial(
    pl.kernel,
    out_type=jax.ShapeDtypeStruct((NC, N), jnp.float32),
    mesh=_mesh,
    scratch_types=[
        pltpu.VMEM((NCHUNK, CHUNK), jnp.int32),
        pltpu.VMEM((128,), jnp.float32),
        pltpu.VMEM_SHARED((N,), jnp.float32),
        pltpu.SemaphoreType.DMA,
    ],
)
def _deg_sc(dst_hbm, zeros1_hbm, out_hbm, dst_v, ones_v, deg_sh, sem):
    c = lax.axis_index("c")
    s = lax.axis_index("s")
    for i in range(128 // 16):
        ones_v[pl.ds(i * 16, 16)] = jnp.ones((16,), jnp.float32)

    @pl.when(s == 0)
    def _zero():
        pltpu.sync_copy(zeros1_hbm, deg_sh)

    plsc.subcore_barrier()
    base = (c * NS + s) * NCHUNK
    pltpu.sync_copy(dst_hbm.at[pl.ds(base, NCHUNK)], dst_v)

    def body(j, carry):
        pltpu.async_copy(ones_v.at[pl.ds(0, CHUNK)], deg_sh.at[dst_v.at[j]],
                         sem, add=True)

        @pl.when(j >= DEG_Q)
        def _pace():
            pltpu.make_async_copy(
                ones_v.at[pl.ds(0, CHUNK)], deg_sh.at[dst_v.at[j]], sem).wait()

        return carry

    lax.fori_loop(0, NCHUNK, body, 0)
    for _ in range(DEG_Q):
        pltpu.make_async_copy(
            ones_v.at[pl.ds(0, CHUNK)], deg_sh.at[dst_v.at[0]], sem).wait()
    plsc.subcore_barrier()

    @pl.when(s == 0)
    def _out():
        pltpu.sync_copy(deg_sh, out_hbm.at[c])


def _edge_phase(table, src_v, dst_v, rows_v, agg_sh, gsems, ssems):
    for b in range(NBUF):
        pltpu.async_copy(table.at[src_v.at[b]], rows_v.at[b], gsems[b])

    def body(o, carry):
        for b in range(NBUF):
            m = o * NBUF + b
            pltpu.make_async_copy(
                table.at[src_v.at[m]], rows_v.at[b], gsems[b]).wait()
            pltpu.async_copy(
                rows_v.at[b], agg_sh.at[dst_v.at[m]], ssems[b], add=True)

            @pl.when(o < NCHUNK // NBUF - 1)
            def _next():
                pltpu.make_async_copy(
                    rows_v.at[b], agg_sh.at[dst_v.at[m]], ssems[b]).wait()
                pltpu.async_copy(
                    table.at[src_v.at[m + NBUF]], rows_v.at[b], gsems[b])

        return carry

    lax.fori_loop(0, NCHUNK // NBUF, body, 0)
    for b in range(NBUF):
        m = NCHUNK - NBUF + b
        pltpu.make_async_copy(
            rows_v.at[b], agg_sh.at[dst_v.at[m]], ssems[b]).wait()


@functools.partial(
    pl.kernel,
    out_type=(
        jax.ShapeDtypeStruct((NC, N, HID), jnp.float32),
        jax.ShapeDtypeStruct((NC, N, HID), jnp.float32),
    ),
    mesh=_mesh,
    scratch_types=[
        pltpu.VMEM((SLAB, HID), jnp.float32),
        pltpu.VMEM((SLAB,), jnp.float32),
        pltpu.VMEM((SLAB,), jnp.float32),
        pltpu.VMEM((NCHUNK, CHUNK), jnp.int32),
        pltpu.VMEM((NCHUNK, CHUNK), jnp.int32),
        pltpu.VMEM((NBUF, CHUNK, HID), jnp.float32),
        pltpu.VMEM_SHARED((N, HID), jnp.float32),
        [pltpu.SemaphoreType.DMA] * NBUF,
        [pltpu.SemaphoreType.DMA] * NBUF,
    ],
    compiler_params=_sc_params,
)
def _agg1_sc(hbar_hbm, degp_hbm, src_hbm, dst_hbm, zeros2_hbm,
             hs1d_hbm, out_hbm,
             hb_v, d0_v, d1_v, src_v, dst_v, rows_v, agg_sh, gsems, ssems):
    c = lax.axis_index("c")
    s = lax.axis_index("s")
    rbase = s * STRIDE
    base = (c * NS + s) * NCHUNK
    pltpu.sync_copy(src_hbm.at[pl.ds(base, NCHUNK)], src_v)
    pltpu.sync_copy(dst_hbm.at[pl.ds(base, NCHUNK)], dst_v)
    pltpu.sync_copy(zeros2_hbm.at[pl.ds(rbase, SLAB)],
                    agg_sh.at[pl.ds(rbase, SLAB)])
    pltpu.sync_copy(hbar_hbm.at[pl.ds(rbase, SLAB)], hb_v)
    pltpu.sync_copy(degp_hbm.at[0, pl.ds(rbase, SLAB)], d0_v)
    pltpu.sync_copy(degp_hbm.at[1, pl.ds(rbase, SLAB)], d1_v)

    def blk(b, carry):
        o = b * 16
        deg = d0_v[pl.ds(o, 16)] + d1_v[pl.ds(o, 16)] + 1.0
        dis = _rsqrt16(deg)
        for i in range(16):
            hb_v[o + i] = hb_v[o + i] * dis[i]
        return carry

    lax.fori_loop(0, NVB, blk, 0)
    pltpu.sync_copy(hb_v, hs1d_hbm.at[c, pl.ds(rbase, SLAB)])
    plsc.subcore_barrier()

    _edge_phase(hs1d_hbm.at[c], src_v, dst_v, rows_v, agg_sh, gsems, ssems)
    plsc.subcore_barrier()
    pltpu.sync_copy(agg_sh.at[pl.ds(rbase, SLAB)],
                    out_hbm.at[c, pl.ds(rbase, SLAB)])


@functools.partial(
    pl.kernel,
    out_type=(
        jax.ShapeDtypeStruct((NC, N, HID), jnp.float32),
        jax.ShapeDtypeStruct((NC, N, HID), jnp.float32),
    ),
    mesh=_mesh,
    scratch_types=[
        pltpu.VMEM((SLAB, HID), jnp.float32),
        pltpu.VMEM((SLAB, HID), jnp.float32),
        pltpu.VMEM((SLAB, HID), jnp.float32),
        pltpu.VMEM((SLAB,), jnp.float32),
        pltpu.VMEM((SLAB,), jnp.float32),
        pltpu.VMEM((HID,), jnp.float32),
        pltpu.VMEM((NCHUNK, CHUNK), jnp.int32),
        pltpu.VMEM((NCHUNK, CHUNK), jnp.int32),
        pltpu.VMEM((NBUF, CHUNK, HID), jnp.float32),
        pltpu.VMEM_SHARED((N, HID), jnp.float32),
        [pltpu.SemaphoreType.DMA] * NBUF,
        [pltpu.SemaphoreType.DMA] * NBUF,
    ],
    compiler_params=_sc_params,
)
def _agg2_sc(agg1_hbm, hs1d_hbm, degp_hbm, b1_hbm, src_hbm, dst_hbm,
             zeros2_hbm, out_hbm, hs2d_hbm,
             p0_v, p1_v, hs1_v, d0_v, d1_v, b1_v, src_v, dst_v, rows_v,
             agg_sh, gsems, ssems):
    c = lax.axis_index("c")
    s = lax.axis_index("s")
    rbase = s * STRIDE
    base = (c * NS + s) * NCHUNK
    pltpu.sync_copy(src_hbm.at[pl.ds(base, NCHUNK)], src_v)
    pltpu.sync_copy(dst_hbm.at[pl.ds(base, NCHUNK)], dst_v)
    pltpu.sync_copy(zeros2_hbm.at[pl.ds(rbase, SLAB)],
                    agg_sh.at[pl.ds(rbase, SLAB)])
    pltpu.sync_copy(agg1_hbm.at[0, pl.ds(rbase, SLAB)], p0_v)
    pltpu.sync_copy(agg1_hbm.at[1, pl.ds(rbase, SLAB)], p1_v)
    pltpu.sync_copy(hs1d_hbm.at[c, pl.ds(rbase, SLAB)], hs1_v)
    pltpu.sync_copy(degp_hbm.at[0, pl.ds(rbase, SLAB)], d0_v)
    pltpu.sync_copy(degp_hbm.at[1, pl.ds(rbase, SLAB)], d1_v)
    pltpu.sync_copy(b1_hbm, b1_v)
    b1r = b1_v[...]

    def blk(b, carry):
        o = b * 16
        deg = d0_v[pl.ds(o, 16)] + d1_v[pl.ds(o, 16)] + 1.0
        dis = _rsqrt16(deg)
        for i in range(16):
            n = o + i
            row = (p0_v[n] + p1_v[n] + hs1_v[n]) * dis[i] + b1r
            p0_v[n] = jnp.maximum(row, 0.0) * dis[i]
        return carry

    lax.fori_loop(0, NVB, blk, 0)
    pltpu.sync_copy(p0_v, hs2d_hbm.at[c, pl.ds(rbase, SLAB)])
    plsc.subcore_barrier()

    _edge_phase(hs2d_hbm.at[c], src_v, dst_v, rows_v, agg_sh, gsems, ssems)
    plsc.subcore_barrier()

    pltpu.sync_copy(agg_sh.at[pl.ds(rbase, SLAB)], hs1_v)

    def qblk(b, carry):
        o = b * 16
        deg = d0_v[pl.ds(o, 16)] + d1_v[pl.ds(o, 16)] + 1.0
        dis = _rsqrt16(deg)
        for i in range(16):
            n = o + i
            hs1_v[n] = (hs1_v[n] + 0.5 * p0_v[n]) * dis[i]
        return carry

    lax.fori_loop(0, NVB, qblk, 0)
    pltpu.sync_copy(hs1_v, out_hbm.at[c, pl.ds(rbase, SLAB)])


def _mm_body(x_ref, w1_ref, hb_ref):
    hb_ref[...] = jnp.dot(x_ref[...], w1_ref[...],
                          preferred_element_type=jnp.float32)


def _tc_mm(x, W1):
    return pl.pallas_call(
        _mm_body,
        out_shape=jax.ShapeDtypeStruct((N, HID), jnp.float32),
    )(x, W1)


def _out_body(q_ref, w2_ref, b2_ref, out_ref):
    a = q_ref[0] + q_ref[1]
    o = jnp.dot(a, w2_ref[...], preferred_element_type=jnp.float32)
    o = o + b2_ref[...]
    m = jnp.max(o, axis=1, keepdims=True)
    e = jnp.exp(o - m)
    lse = jnp.log(jnp.sum(e, axis=1, keepdims=True))
    out_ref[...] = (o - m) - lse


def _tc_out(q, W2, b2):
    return pl.pallas_call(
        _out_body,
        out_shape=jax.ShapeDtypeStruct((N, NUM_CLASSES), jnp.float32),
    )(q, W2, b2)


def kernel(x, edge_index, W1, b1, W2, b2):
    src2d = edge_index[0].reshape(E // CHUNK, CHUNK)
    dst2d = edge_index[1].reshape(E // CHUNK, CHUNK)
    zeros1 = jnp.zeros((N,), jnp.float32)
    zeros2 = jnp.zeros((N, HID), jnp.float32)

    hbar = _tc_mm(x, W1)
    degp = _deg_sc(dst2d, zeros1)
    hs1d, agg1 = _agg1_sc(hbar, degp, src2d, dst2d, zeros2)
    q, _hs2d = _agg2_sc(agg1, hs1d, degp, b1, src2d, dst2d, zeros2)
    return _tc_out(q, W2, b2.reshape(1, NUM_CLASSES))
